# baseline (speedup 1.0000x reference)
import functools

import jax
import jax.numpy as jnp
from jax import lax
from jax.experimental import pallas as pl
from jax.experimental.pallas import tpu as pltpu
from jax.experimental.pallas import tpu_sc as plsc

NUM_ITEM = 1000000
EMBED_DIM = 32
BATCH = 16384

_NC = 2
_NS = 16
_NW = _NC * _NS
_B_PER_W = BATCH // _NW
_CHUNK = 128
_NCHUNK = _B_PER_W // _CHUNK
_L = 16

_mesh = plsc.VectorSubcoreMesh(core_axis_name="c", subcore_axis_name="s")


@functools.partial(
    pl.kernel,
    mesh=_mesh,
    out_type=jax.ShapeDtypeStruct((EMBED_DIM, BATCH), jnp.float32),
    scratch_types=[
        pltpu.VMEM((_NCHUNK, _CHUNK), jnp.int32),
        pltpu.VMEM((_B_PER_W, EMBED_DIM), jnp.float32),
        pltpu.VMEM((EMBED_DIM, _B_PER_W), jnp.float32),
        pltpu.SemaphoreType.DMA,
    ],
    compiler_params=pltpu.CompilerParams(
        use_tc_tiling_on_sc=False, needs_layout_passes=False
    ),
)
def _gather_kernel(table_hbm, idx_hbm, out_hbm, idx_v, rows_v, outt_v, sem):
    wid = lax.axis_index("s") * _NC + lax.axis_index("c")
    base = wid * _B_PER_W
    pltpu.sync_copy(idx_hbm.at[wid], idx_v)

    copies = []
    for t in range(_NCHUNK):
        copies.append(
            pltpu.async_copy(
                table_hbm.at[idx_v.at[t]],
                rows_v.at[pl.ds(t * _CHUNK, _CHUNK)],
                sem,
            )
        )
    for c in copies:
        c.wait()

    def block_body(b, _):
        jvec = lax.iota(jnp.int32, _L) + b * _L
        for c in range(EMBED_DIM):
            cvec = jnp.full((_L,), c, jnp.int32)
            vals = plsc.load_gather(rows_v, [jvec, cvec])
            outt_v[c, pl.ds(b * _L, _L)] = vals
        return 0

    lax.fori_loop(0, _B_PER_W // _L, block_body, 0)
    pltpu.sync_copy(outt_v, out_hbm.at[:, pl.ds(base, _B_PER_W)])


def kernel(item_fea, embedding_itemId):
    idx = item_fea[:, 0].astype(jnp.int32).reshape(_NW, _NCHUNK, _CHUNK)
    out_t = _gather_kernel(embedding_itemId, idx)
    return out_t.T


# speedup vs baseline: 1.0007x; 1.0007x over previous
import functools

import jax
import jax.numpy as jnp
from jax import lax
from jax.experimental import pallas as pl
from jax.experimental.pallas import tpu as pltpu
from jax.experimental.pallas import tpu_sc as plsc

NUM_ITEM = 1000000
EMBED_DIM = 32
BATCH = 16384

_NC = 2
_NS = 16
_NW = _NC * _NS
_B_PER_W = BATCH // _NW
_CHUNK = 128
_NCHUNK = _B_PER_W // _CHUNK
_L = 16

_mesh = plsc.VectorSubcoreMesh(core_axis_name="c", subcore_axis_name="s")


@functools.partial(
    pl.kernel,
    mesh=_mesh,
    out_type=jax.ShapeDtypeStruct((EMBED_DIM, BATCH), jnp.float32),
    scratch_types=[
        pltpu.VMEM((_B_PER_W,), jnp.int32),
        pltpu.VMEM((_B_PER_W, EMBED_DIM), jnp.float32),
        pltpu.VMEM((EMBED_DIM, _B_PER_W), jnp.float32),
        pltpu.SemaphoreType.DMA,
    ],
    compiler_params=pltpu.CompilerParams(
        use_tc_tiling_on_sc=False, needs_layout_passes=False
    ),
)
def _gather_kernel(table_hbm, idx_hbm, out_hbm, idx_v, rows_v, outt_v, sem):
    wid = lax.axis_index("s") * _NC + lax.axis_index("c")
    base = wid * _B_PER_W
    pltpu.sync_copy(idx_hbm.at[pl.ds(base, _B_PER_W)], idx_v)

    copies = []
    for t in range(_NCHUNK):
        copies.append(
            pltpu.async_copy(
                table_hbm.at[idx_v.at[pl.ds(t * _CHUNK, _CHUNK)]],
                rows_v.at[pl.ds(t * _CHUNK, _CHUNK)],
                sem,
            )
        )
    for c in copies:
        c.wait()

    def block_body(b, _):
        jvec = lax.iota(jnp.int32, _L) + b * _L
        for c in range(EMBED_DIM):
            cvec = jnp.full((_L,), c, jnp.int32)
            vals = plsc.load_gather(rows_v, [jvec, cvec])
            outt_v[c, pl.ds(b * _L, _L)] = vals
        return 0

    lax.fori_loop(0, _B_PER_W // _L, block_body, 0)
    pltpu.sync_copy(outt_v, out_hbm.at[:, pl.ds(base, _B_PER_W)])


def kernel(item_fea, embedding_itemId):
    idx = item_fea[:, 0].astype(jnp.int32)
    out_t = _gather_kernel(embedding_itemId, idx)
    return out_t.T


# speedup vs baseline: 1.6164x; 1.6152x over previous
import functools

import jax
import jax.numpy as jnp
from jax import lax
from jax.experimental import pallas as pl
from jax.experimental.pallas import tpu as pltpu
from jax.experimental.pallas import tpu_sc as plsc

NUM_ITEM = 1000000
EMBED_DIM = 32
BATCH = 16384

_NC = 2
_NS = 16
_NW = _NC * _NS
_B_PER_W = BATCH // _NW
_K = 32

_mesh = plsc.VectorSubcoreMesh(core_axis_name="c", subcore_axis_name="s")


@functools.partial(
    pl.kernel,
    mesh=_mesh,
    out_type=jax.ShapeDtypeStruct((BATCH, EMBED_DIM), jnp.float32),
    scratch_types=[
        pltpu.VMEM((_B_PER_W,), jnp.int32),
        pltpu.VMEM((_B_PER_W, EMBED_DIM), jnp.float32),
        pltpu.SemaphoreType.DMA,
    ],
    compiler_params=pltpu.CompilerParams(
        use_tc_tiling_on_sc=True,
        disable_bounds_checks=True,
        disable_semaphore_checks=True,
    ),
)
def _gather_kernel(table_hbm, idx_hbm, out_hbm, idx_v, rows_v, sem):
    wid = lax.axis_index("s") * _NC + lax.axis_index("c")
    base = wid * _B_PER_W
    pltpu.sync_copy(idx_hbm.at[pl.ds(base, _B_PER_W)], idx_v)

    def batch_body(g, _):
        ivec0 = idx_v[pl.ds(g * _K, 16)]
        ivec1 = idx_v[pl.ds(g * _K + 16, 16)]
        copies = []
        for j in range(_K):
            row = ivec0[j] if j < 16 else ivec1[j - 16]
            copies.append(
                pltpu.async_copy(
                    table_hbm.at[pl.ds(row, 1)],
                    rows_v.at[pl.ds(g * _K + j, 1)],
                    sem,
                )
            )
        for c in copies:
            c.wait()
        return 0

    lax.fori_loop(0, _B_PER_W // _K, batch_body, 0)
    pltpu.sync_copy(rows_v, out_hbm.at[pl.ds(base, _B_PER_W)])


def kernel(item_fea, embedding_itemId):
    idx = item_fea[:, 0].astype(jnp.int32)
    return _gather_kernel(embedding_itemId, idx)
